# Initial kernel scaffold; baseline (speedup 1.0000x reference)
import functools

import jax
import jax.numpy as jnp
from jax import lax
from jax.experimental import pallas as pl
from jax.experimental.pallas import tpu as pltpu
from jax.experimental.pallas import tpu_sc as plsc

NC = 2
NS = 16
L = 16
NW = NC * NS
B = 8192


def _spmv_grid(n, nnz):
    nslice = n // NS
    j_full = nnz // B
    mesh = plsc.VectorSubcoreMesh(core_axis_name="c", subcore_axis_name="s")

    def process_chunk(x_v, vals_v, cols_v, rows_v, prod_v, y_sh):
        def gm(i, _):
            c16 = cols_v[pl.ds(i * L, L)]
            v16 = vals_v[pl.ds(i * L, L)]
            xv = plsc.load_gather(x_v, [c16])
            prod_v[pl.ds(i * L, L)] = xv * v16
            return 0
        lax.fori_loop(0, B // L, gm, 0)
        pltpu.sync_copy(prod_v, y_sh.at[rows_v], add=True)

    @functools.partial(
        pl.kernel,
        out_type=jax.ShapeDtypeStruct((NC, n), jnp.float32),
        mesh=mesh,
        scratch_types=[
            pltpu.VMEM((n,), jnp.float32),
            pltpu.VMEM((B,), jnp.float32),
            pltpu.VMEM((B,), jnp.int32),
            pltpu.VMEM((B,), jnp.int32),
            pltpu.VMEM((B,), jnp.float32),
            pltpu.VMEM_SHARED((n,), jnp.float32),
        ],
    )
    def k(vals_hbm, rows_hbm, cols_hbm, x_hbm, tval, trow, tcol, out,
          x_v, vals_v, cols_v, rows_v, prod_v, y_sh):
        cid = lax.axis_index("c")
        sid = lax.axis_index("s")
        wid = sid * NC + cid

        def z(i, _):
            prod_v[pl.ds(i * L, L)] = jnp.zeros((L,), jnp.float32)
            return 0
        lax.fori_loop(0, nslice // L, z, 0)
        pltpu.sync_copy(prod_v.at[pl.ds(0, nslice)],
                        y_sh.at[pl.ds(sid * nslice, nslice)])
        pltpu.sync_copy(x_hbm, x_v)
        plsc.subcore_barrier()

        nsub = (j_full + NW - 1 - wid) // NW

        def sub(i, _):
            base = (i * NW + wid) * B
            pltpu.sync_copy(vals_hbm.at[pl.ds(base, B)], vals_v)
            pltpu.sync_copy(cols_hbm.at[pl.ds(base, B)], cols_v)
            pltpu.sync_copy(rows_hbm.at[pl.ds(base, B)], rows_v)
            process_chunk(x_v, vals_v, cols_v, rows_v, prod_v, y_sh)
            return 0
        lax.fori_loop(0, nsub, sub, 0)

        @pl.when(wid == NW - 1)
        def _():
            pltpu.sync_copy(tval, vals_v)
            pltpu.sync_copy(tcol, cols_v)
            pltpu.sync_copy(trow, rows_v)
            process_chunk(x_v, vals_v, cols_v, rows_v, prod_v, y_sh)

        plsc.subcore_barrier()
        pltpu.sync_copy(y_sh.at[pl.ds(sid * nslice, nslice)],
                        out.at[cid, pl.ds(sid * nslice, nslice)])

    return k


def _combine_body(p_ref, o_ref):
    o_ref[...] = p_ref[0, :] + p_ref[1, :]


def kernel(H_vals, H_rows, H_cols, x):
    n = x.shape[0]
    nnz = H_vals.shape[0]
    j_full = nnz // B
    tail = nnz - j_full * B
    rows = H_rows.astype(jnp.int32)
    cols = H_cols.astype(jnp.int32)
    tval = jnp.zeros((B,), jnp.float32).at[:tail].set(H_vals[j_full * B:])
    trow = jnp.zeros((B,), jnp.int32).at[:tail].set(rows[j_full * B:])
    tcol = jnp.zeros((B,), jnp.int32).at[:tail].set(cols[j_full * B:])
    partial = _spmv_grid(n, nnz)(H_vals, rows, cols, x, tval, trow, tcol)
    y = pl.pallas_call(
        _combine_body,
        out_shape=jax.ShapeDtypeStruct((n,), jnp.float32),
    )(partial)
    return y


# speedup vs baseline: 237.4148x; 237.4148x over previous
import functools

import jax
import jax.numpy as jnp
from jax import lax
from jax.experimental import pallas as pl
from jax.experimental.pallas import tpu as pltpu
from jax.experimental.pallas import tpu_sc as plsc

NC = 2
NS = 16
L = 16
NW = NC * NS
B = 8192


def _spmv_grid(n, nnz):
    nslice = n // NS
    j_full = nnz // B
    mesh = plsc.VectorSubcoreMesh(core_axis_name="c", subcore_axis_name="s")

    def process_chunk(x_v, vals_v, cols_v, rows_v, prod_v, y_sh):
        def gm(i, _):
            c16 = cols_v[pl.ds(i * L, L)]
            v16 = vals_v[pl.ds(i * L, L)]
            xv = plsc.load_gather(x_v, [c16])
            prod_v[pl.ds(i * L, L)] = xv * v16
            return 0
        lax.fori_loop(0, B // L, gm, 0)
        pltpu.sync_copy(prod_v, y_sh.at[rows_v], add=True)

    @functools.partial(
        pl.kernel,
        out_type=jax.ShapeDtypeStruct((NC, n), jnp.float32),
        mesh=mesh,
        compiler_params=pltpu.CompilerParams(needs_layout_passes=False),
        scratch_types=[
            pltpu.VMEM((n,), jnp.float32),
            pltpu.VMEM((B,), jnp.float32),
            pltpu.VMEM((B,), jnp.int32),
            pltpu.VMEM((B,), jnp.int32),
            pltpu.VMEM((B,), jnp.float32),
            pltpu.VMEM_SHARED((n,), jnp.float32),
        ],
    )
    def k(vals_hbm, rows_hbm, cols_hbm, x_hbm, tval, trow, tcol, out,
          x_v, vals_v, cols_v, rows_v, prod_v, y_sh):
        cid = lax.axis_index("c")
        sid = lax.axis_index("s")
        wid = sid * NC + cid

        def z(i, _):
            prod_v[pl.ds(i * L, L)] = jnp.zeros((L,), jnp.float32)
            return 0
        lax.fori_loop(0, nslice // L, z, 0)
        pltpu.sync_copy(prod_v.at[pl.ds(0, nslice)],
                        y_sh.at[pl.ds(sid * nslice, nslice)])
        pltpu.sync_copy(x_hbm, x_v)
        plsc.subcore_barrier()

        nsub = (j_full + NW - 1 - wid) // NW

        def sub(i, _):
            base = (i * NW + wid) * B
            pltpu.sync_copy(vals_hbm.at[pl.ds(base, B)], vals_v)
            pltpu.sync_copy(cols_hbm.at[pl.ds(base, B)], cols_v)
            pltpu.sync_copy(rows_hbm.at[pl.ds(base, B)], rows_v)
            process_chunk(x_v, vals_v, cols_v, rows_v, prod_v, y_sh)
            return 0
        lax.fori_loop(0, nsub, sub, 0)

        @pl.when(wid == NW - 1)
        def _():
            pltpu.sync_copy(tval, vals_v)
            pltpu.sync_copy(tcol, cols_v)
            pltpu.sync_copy(trow, rows_v)
            process_chunk(x_v, vals_v, cols_v, rows_v, prod_v, y_sh)

        plsc.subcore_barrier()
        pltpu.sync_copy(y_sh.at[pl.ds(sid * nslice, nslice)],
                        out.at[cid, pl.ds(sid * nslice, nslice)])

    return k


def _combine_body(p_ref, o_ref):
    o_ref[...] = p_ref[0, :] + p_ref[1, :]


def kernel(H_vals, H_rows, H_cols, x):
    n = x.shape[0]
    nnz = H_vals.shape[0]
    j_full = nnz // B
    tail = nnz - j_full * B
    rows = H_rows.astype(jnp.int32)
    cols = H_cols.astype(jnp.int32)
    tval = jnp.zeros((B,), jnp.float32).at[:tail].set(H_vals[j_full * B:])
    trow = jnp.zeros((B,), jnp.int32).at[:tail].set(rows[j_full * B:])
    tcol = jnp.zeros((B,), jnp.int32).at[:tail].set(cols[j_full * B:])
    partial = _spmv_grid(n, nnz)(H_vals, rows, cols, x, tval, trow, tcol)
    y = pl.pallas_call(
        _combine_body,
        out_shape=jax.ShapeDtypeStruct((n,), jnp.float32),
    )(partial)
    return y


# speedup vs baseline: 396.6638x; 1.6708x over previous
import functools

import jax
import jax.numpy as jnp
from jax import lax
from jax.experimental import pallas as pl
from jax.experimental.pallas import tpu as pltpu
from jax.experimental.pallas import tpu_sc as plsc

NC = 2
NS = 16
L = 16
NW = NC * NS
B = 7552


def _spmv_grid(n, nnz):
    nslice = n // NS
    j_full = nnz // B
    mesh = plsc.VectorSubcoreMesh(core_axis_name="c", subcore_axis_name="s")

    @functools.partial(
        pl.kernel,
        out_type=jax.ShapeDtypeStruct((NC, n), jnp.float32),
        mesh=mesh,
        compiler_params=pltpu.CompilerParams(needs_layout_passes=False),
        scratch_types=[
            pltpu.VMEM((n,), jnp.float32),
            pltpu.VMEM((B,), jnp.float32),
            pltpu.VMEM((B,), jnp.float32),
            pltpu.VMEM((B,), jnp.int32),
            pltpu.VMEM((B,), jnp.int32),
            pltpu.VMEM((B,), jnp.int32),
            pltpu.VMEM((B,), jnp.int32),
            pltpu.VMEM((B,), jnp.float32),
            pltpu.VMEM((B,), jnp.float32),
            pltpu.VMEM_SHARED((n,), jnp.float32),
            pltpu.SemaphoreType.DMA((2,)),
            pltpu.SemaphoreType.DMA((2,)),
        ],
    )
    def k(vals_hbm, rows_hbm, cols_hbm, x_hbm, tval, trow, tcol, out,
          x_v, vals0, vals1, cols0, cols1, rows0, rows1, prod0, prod1,
          y_sh, dsem, ssem):
        cid = lax.axis_index("c")
        sid = lax.axis_index("s")
        wid = sid * NC + cid
        vals_v = (vals0, vals1)
        cols_v = (cols0, cols1)
        rows_v = (rows0, rows1)
        prod_v = (prod0, prod1)

        def issue_in(b, r):
            base = (r * NW + wid) * B
            pltpu.async_copy(vals_hbm.at[pl.ds(base, B)], vals_v[b],
                             dsem.at[b])
            pltpu.async_copy(cols_hbm.at[pl.ds(base, B)], cols_v[b],
                             dsem.at[b])
            pltpu.async_copy(rows_hbm.at[pl.ds(base, B)], rows_v[b],
                             dsem.at[b])

        def wait_in(b, r):
            base = (r * NW + wid) * B
            pltpu.make_async_copy(vals_hbm.at[pl.ds(base, B)], vals_v[b],
                                  dsem.at[b]).wait()
            pltpu.make_async_copy(cols_hbm.at[pl.ds(base, B)], cols_v[b],
                                  dsem.at[b]).wait()
            pltpu.make_async_copy(rows_hbm.at[pl.ds(base, B)], rows_v[b],
                                  dsem.at[b]).wait()

        def compute(b):
            def gm(i, _):
                c16 = cols_v[b][pl.ds(i * L, L)]
                v16 = vals_v[b][pl.ds(i * L, L)]
                xv = plsc.load_gather(x_v, [c16])
                prod_v[b][pl.ds(i * L, L)] = xv * v16
                return 0
            lax.fori_loop(0, B // L, gm, 0)

        def issue_scatter(b):
            pltpu.async_copy(prod_v[b], y_sh.at[rows_v[b]], ssem.at[b],
                             add=True)

        def wait_scatter(b):
            pltpu.make_async_copy(prod_v[b], y_sh.at[rows_v[b]],
                                  ssem.at[b]).wait()

        def z(i, _):
            prod0[pl.ds(i * L, L)] = jnp.zeros((L,), jnp.float32)
            return 0
        lax.fori_loop(0, nslice // L, z, 0)
        pltpu.sync_copy(prod0.at[pl.ds(0, nslice)],
                        y_sh.at[pl.ds(sid * nslice, nslice)])
        pltpu.sync_copy(x_hbm, x_v)
        plsc.subcore_barrier()

        nsub = (j_full + NW - 1 - wid) // NW

        @pl.when(nsub > 0)
        def _():
            issue_in(0, 0)

        @pl.loop(0, nsub, step=2)
        def _(outer):
            for b in range(2):
                r = outer + b

                @pl.when(r < nsub)
                def _(r=r, b=b):
                    wait_in(b, r)
                    compute(b)
                    issue_scatter(b)

                    @pl.when(r + 1 < nsub)
                    def _(r=r, b=b):
                        @pl.when(r >= 1)
                        def _(b=b):
                            wait_scatter(1 - b)
                        issue_in(1 - b, r + 1)

        @pl.when(nsub >= 1)
        def _():
            wait_scatter(0)

        @pl.when(nsub >= 2)
        def _():
            wait_scatter(1)

        @pl.when(wid == NW - 1)
        def _():
            pltpu.sync_copy(tval, vals0)
            pltpu.sync_copy(tcol, cols0)
            pltpu.sync_copy(trow, rows0)
            compute(0)
            pltpu.sync_copy(prod0, y_sh.at[rows0], add=True)

        plsc.subcore_barrier()
        pltpu.sync_copy(y_sh.at[pl.ds(sid * nslice, nslice)],
                        out.at[cid, pl.ds(sid * nslice, nslice)])

    return k


def _combine_body(p_ref, o_ref):
    o_ref[...] = p_ref[0, :] + p_ref[1, :]


def kernel(H_vals, H_rows, H_cols, x):
    n = x.shape[0]
    nnz = H_vals.shape[0]
    j_full = nnz // B
    tail = nnz - j_full * B
    rows = H_rows.astype(jnp.int32)
    cols = H_cols.astype(jnp.int32)
    tval = jnp.zeros((B,), jnp.float32).at[:tail].set(H_vals[j_full * B:])
    trow = jnp.zeros((B,), jnp.int32).at[:tail].set(rows[j_full * B:])
    tcol = jnp.zeros((B,), jnp.int32).at[:tail].set(cols[j_full * B:])
    partial = _spmv_grid(n, nnz)(H_vals, rows, cols, x, tval, trow, tcol)
    y = pl.pallas_call(
        _combine_body,
        out_shape=jax.ShapeDtypeStruct((n,), jnp.float32),
    )(partial)
    return y
